# Initial kernel scaffold; baseline (speedup 1.0000x reference)
import jax
import jax.numpy as jnp
from jax.experimental import pallas as pl


def kernel(positions, features, We0, be0, We1, be1, Wx0, bx0, Wx1, bx1, Wh0, bh0, Wh1, bh1, Wf, bf):
    raise NotImplementedError("write your pallas kernel here")


# speedup vs baseline: 20.7295x; 20.7295x over previous
import functools

import jax
import jax.numpy as jnp
from jax.experimental import pallas as pl

_F32 = jnp.float32
_TR = 8


def _silu(z):
    return z * jax.nn.sigmoid(z)


def _prep_kernel(x8_ref, h_ref, wea_ref, web_ref, be0_ref,
                 a3_ref, b_ref, p_ref, q3_ref, x3_ref):
    x = x8_ref[...]
    h = h_ref[...]
    n = x.shape[0]
    a = jnp.dot(h, wea_ref[...], preferred_element_type=_F32)
    b = jnp.dot(h, web_ref[...], preferred_element_type=_F32) + be0_ref[...]
    x3 = x[:, 0:3]
    nx = jnp.sum(x3 * x3, axis=1, keepdims=True)
    ones = jnp.ones((n, 1), _F32)
    zer3 = jnp.zeros((n, 3), _F32)
    a3_ref[...] = a[None]
    b_ref[...] = b
    p_ref[...] = jnp.concatenate([x3, nx, ones, zer3], axis=1)
    q3_ref[...] = jnp.concatenate([-2.0 * x3, ones, nx, zer3], axis=1)[None]
    x3_ref[...] = x[None]


def _edge_kernel(a3_ref, b_ref, p_ref, q3_ref, x3_ref,
                 wsq_ref, we1_ref, be1_ref, wx0_ref, bx0_ref,
                 wx1_ref, bx1_ref,
                 magg_ref, shift_ref, *, n, tr, hdim):
    i0 = pl.program_id(0) * tr
    e = tr * n

    a_part = jnp.broadcast_to(a3_ref[...], (tr, n, hdim)).reshape(e, hdim)
    qt = jnp.broadcast_to(q3_ref[...], (tr, n, 8)).reshape(e, 8)

    grp_col = jax.lax.broadcasted_iota(jnp.int32, (e, tr), 0) // n
    idx_col = jax.lax.broadcasted_iota(jnp.int32, (e, tr), 1)
    gt = (grp_col == idx_col).astype(_F32)
    grp_row = jax.lax.broadcasted_iota(jnp.int32, (tr, e), 1) // n
    idx_row = jax.lax.broadcasted_iota(jnp.int32, (tr, e), 0)
    g = (grp_row == idx_row).astype(_F32)

    b_part = jnp.dot(gt, b_ref[...], preferred_element_type=_F32)
    p_part = jnp.dot(gt, p_ref[...], preferred_element_type=_F32)

    sq = jnp.maximum(jnp.sum(qt * p_part, axis=1, keepdims=True), 0.0)

    m0 = _silu(a_part + b_part + sq * wsq_ref[...])
    m1 = _silu(jnp.dot(m0, we1_ref[...], preferred_element_type=_F32)
               + be1_ref[...])
    av = _silu(jnp.dot(m1, wx0_ref[...], preferred_element_type=_F32)
               + bx0_ref[...])
    coef = jnp.dot(av, wx1_ref[...], preferred_element_type=_F32) + bx1_ref[...]
    scale = coef / (jnp.sqrt(sq) + 1.0)

    e_idx = jax.lax.broadcasted_iota(jnp.int32, (e, 1), 0)
    notdiag = (e_idx % n != e_idx // n + i0).astype(_F32)
    magg_ref[...] = jnp.dot(g, m1 * notdiag, preferred_element_type=_F32)

    xt = jnp.broadcast_to(x3_ref[...], (tr, n, 8)).reshape(e, 8)
    part1 = jnp.dot(g, scale * xt, preferred_element_type=_F32)
    ssum = jnp.dot(g, scale, preferred_element_type=_F32)
    shift_ref[...] = part1 - p_ref[...] * ssum


def _update_kernel(h_ref, x8_ref, magg_ref, shift_ref,
                   wh0a_ref, wh0b_ref, bh0_ref, wh1_ref, bh1_ref,
                   hn_ref, xn_ref, *, n):
    inv = _F32(1.0 / (n - 1))
    h = h_ref[...]
    magg = magg_ref[...] * inv
    t = _silu(jnp.dot(h, wh0a_ref[...], preferred_element_type=_F32)
              + jnp.dot(magg, wh0b_ref[...], preferred_element_type=_F32)
              + bh0_ref[...])
    dh = jnp.dot(t, wh1_ref[...], preferred_element_type=_F32) + bh1_ref[...]
    hn_ref[...] = h + dh
    xs = x8_ref[:, 0:3] + shift_ref[:, 0:3] * inv
    xn_ref[...] = jnp.concatenate([xs, jnp.zeros((n, 5), _F32)], axis=1)


def _update_head_kernel(h_ref, x8_ref, magg_ref, shift_ref, pos8_ref,
                        wh0a_ref, wh0b_ref, bh0_ref, wh1_ref, bh1_ref,
                        wf_ref, bf_ref,
                        hout_ref, vec_ref, *, n):
    inv = _F32(1.0 / (n - 1))
    h = h_ref[...]
    magg = magg_ref[...] * inv
    t = _silu(jnp.dot(h, wh0a_ref[...], preferred_element_type=_F32)
              + jnp.dot(magg, wh0b_ref[...], preferred_element_type=_F32)
              + bh0_ref[...])
    dh = jnp.dot(t, wh1_ref[...], preferred_element_type=_F32) + bh1_ref[...]
    h2 = h + dh
    xs = x8_ref[:, 0:3] + shift_ref[:, 0:3] * inv
    vec3 = xs - pos8_ref[:, 0:3]
    vec_ref[...] = jnp.concatenate([vec3, jnp.zeros((n, 5), _F32)], axis=1)
    z = h2 - jnp.max(h2, axis=1, keepdims=True)
    ez = jnp.exp(z)
    sm = ez / jnp.sum(ez, axis=1, keepdims=True)
    hout_ref[...] = jnp.dot(sm, wf_ref[...], preferred_element_type=_F32) \
        + bf_ref[...]


def kernel(positions, features, We0, be0, We1, be1, Wx0, bx0, Wx1, bx1,
           Wh0, bh0, Wh1, bh1, Wf, bf):
    n = positions.shape[0]
    hdim = features.shape[-1]
    nb = We0.shape[0]
    tr = _TR
    nt = n // tr

    h = features[:, 0, :].astype(_F32)
    x8 = jnp.pad(positions[:, 0, :].astype(_F32), ((0, 0), (0, 5)))
    pos8 = x8

    prep_call = pl.pallas_call(
        _prep_kernel,
        out_shape=[
            jax.ShapeDtypeStruct((1, n, hdim), _F32),
            jax.ShapeDtypeStruct((n, hdim), _F32),
            jax.ShapeDtypeStruct((n, 8), _F32),
            jax.ShapeDtypeStruct((1, n, 8), _F32),
            jax.ShapeDtypeStruct((1, n, 8), _F32),
        ],
    )

    full = lambda t: (0, 0)
    edge_call = pl.pallas_call(
        functools.partial(_edge_kernel, n=n, tr=tr, hdim=hdim),
        grid=(nt,),
        in_specs=[
            pl.BlockSpec((1, n, hdim), lambda t: (0, 0, 0)),
            pl.BlockSpec((tr, hdim), lambda t: (t, 0)),
            pl.BlockSpec((tr, 8), lambda t: (t, 0)),
            pl.BlockSpec((1, n, 8), lambda t: (0, 0, 0)),
            pl.BlockSpec((1, n, 8), lambda t: (0, 0, 0)),
            pl.BlockSpec((1, hdim), full),
            pl.BlockSpec((hdim, hdim), full),
            pl.BlockSpec((1, hdim), full),
            pl.BlockSpec((hdim, hdim), full),
            pl.BlockSpec((1, hdim), full),
            pl.BlockSpec((hdim, 1), full),
            pl.BlockSpec((1, 1), full),
        ],
        out_specs=[
            pl.BlockSpec((tr, hdim), lambda t: (t, 0)),
            pl.BlockSpec((tr, 8), lambda t: (t, 0)),
        ],
        out_shape=[
            jax.ShapeDtypeStruct((n, hdim), _F32),
            jax.ShapeDtypeStruct((n, 8), _F32),
        ],
    )

    update_call = pl.pallas_call(
        functools.partial(_update_kernel, n=n),
        out_shape=[
            jax.ShapeDtypeStruct((n, hdim), _F32),
            jax.ShapeDtypeStruct((n, 8), _F32),
        ],
    )

    head_call = pl.pallas_call(
        functools.partial(_update_head_kernel, n=n),
        out_shape=[
            jax.ShapeDtypeStruct((n, hdim), _F32),
            jax.ShapeDtypeStruct((n, 8), _F32),
        ],
    )

    h_out = None
    vec8 = None
    for b in range(nb):
        wea = We0[b, :hdim]
        web = We0[b, hdim:2 * hdim]
        wsq = We0[b, 2 * hdim:2 * hdim + 1]
        a3, bmat, pmat, q3, x3a = prep_call(x8, h, wea, web, be0[b][None])
        magg, shift = edge_call(a3, bmat, pmat, q3, x3a,
                                wsq, We1[b], be1[b][None], Wx0[b],
                                bx0[b][None], Wx1[b], bx1[b][None])
        if b < nb - 1:
            h, x8 = update_call(h, x8, magg, shift,
                                Wh0[b, :hdim], Wh0[b, hdim:], bh0[b][None],
                                Wh1[b], bh1[b][None])
        else:
            h_out, vec8 = head_call(h, x8, magg, shift, pos8,
                                    Wh0[b, :hdim], Wh0[b, hdim:],
                                    bh0[b][None], Wh1[b], bh1[b][None],
                                    Wf, bf[None])

    vectors = vec8[:, 0:3][:, None, :]
    return vectors, h_out


# speedup vs baseline: 33.9396x; 1.6373x over previous
import functools

import jax
import jax.numpy as jnp
from jax.experimental import pallas as pl

_F32 = jnp.float32
_TR = 8


def _silu(z):
    return z * jax.nn.sigmoid(z)


def _prep_kernel(x8_ref, h_ref, wea_ref, web_ref, be0_ref,
                 a_ref, b_ref, p_ref, q_ref, x_ref):
    x = x8_ref[...]
    h = h_ref[...]
    n = x.shape[0]
    a_ref[...] = jnp.dot(h, wea_ref[...], preferred_element_type=_F32)
    b_ref[...] = jnp.dot(h, web_ref[...], preferred_element_type=_F32) \
        + be0_ref[...]
    x3 = x[:, 0:3]
    nx = jnp.sum(x3 * x3, axis=1, keepdims=True)
    ones = jnp.ones((n, 1), _F32)
    zer3 = jnp.zeros((n, 3), _F32)
    p_ref[...] = jnp.concatenate([x3, nx, ones, zer3], axis=1)
    q_ref[...] = jnp.concatenate([-2.0 * x3, ones, nx, zer3], axis=1)
    x_ref[...] = x


def _edge_kernel(a2_ref, a_ref, b_ref, p_ref, q2_ref, x2_ref,
                 gt_ref, g_ref,
                 we1bd_ref, be1d_ref, wx0bd_ref, bx0d_ref,
                 wx1sel_ref, bx1_ref, selw_ref, sel16_ref,
                 we1_ref, be1_ref,
                 magg_ref, shift_ref, *, n, tr, hdim):
    n2 = n // 2
    e2 = tr * n2

    a_part = jnp.broadcast_to(a2_ref[...], (tr, n2, 2 * hdim)).reshape(
        e2, 2 * hdim)
    qt = jnp.broadcast_to(q2_ref[...], (tr, n2, 16)).reshape(e2, 16)

    gt = gt_ref[...]
    g = g_ref[...]

    bt = b_ref[...]
    pt = p_ref[...]
    b2t = jnp.concatenate([bt, bt], axis=1)
    p2t = jnp.concatenate([pt, pt], axis=1)

    b_part = jnp.dot(gt, b2t, preferred_element_type=_F32)
    p_part = jnp.dot(gt, p2t, preferred_element_type=_F32)
    pq = qt * p_part

    sq16 = jnp.maximum(
        jnp.dot(pq, sel16_ref[...], preferred_element_type=_F32), 0.0)
    sqw = jnp.dot(pq, selw_ref[...], preferred_element_type=_F32)

    m0 = _silu(a_part + b_part + sqw)
    m1 = _silu(jnp.dot(m0, we1bd_ref[...], preferred_element_type=_F32)
               + be1d_ref[...])
    av = _silu(jnp.dot(m1, wx0bd_ref[...], preferred_element_type=_F32)
               + bx0d_ref[...])
    coef16 = jnp.dot(av, wx1sel_ref[...], preferred_element_type=_F32) \
        + bx1_ref[...]
    scale16 = coef16 / (jnp.sqrt(sq16) + 1.0)

    maggp = jnp.dot(g, m1, preferred_element_type=_F32)
    m0d = _silu(a_ref[...] + bt)
    m1d = _silu(jnp.dot(m0d, we1_ref[...], preferred_element_type=_F32)
                + be1_ref[...])
    magg_ref[...] = maggp[:, :hdim] + maggp[:, hdim:] - m1d

    xt = jnp.broadcast_to(x2_ref[...], (tr, n2, 16)).reshape(e2, 16)
    part1p = jnp.dot(g, scale16 * xt, preferred_element_type=_F32)
    part1 = part1p[:, 0:8] + part1p[:, 8:16]
    ssump = jnp.dot(g, scale16, preferred_element_type=_F32)
    ssum = ssump[:, 0:1] + ssump[:, 8:9]
    shift_ref[...] = part1 - pt * ssum


def _update_kernel(h_ref, x8_ref, magg_ref, shift_ref,
                   wh0a_ref, wh0b_ref, bh0_ref, wh1_ref, bh1_ref,
                   hn_ref, xn_ref, *, n):
    inv = _F32(1.0 / (n - 1))
    h = h_ref[...]
    magg = magg_ref[...] * inv
    t = _silu(jnp.dot(h, wh0a_ref[...], preferred_element_type=_F32)
              + jnp.dot(magg, wh0b_ref[...], preferred_element_type=_F32)
              + bh0_ref[...])
    dh = jnp.dot(t, wh1_ref[...], preferred_element_type=_F32) + bh1_ref[...]
    hn_ref[...] = h + dh
    xs = x8_ref[:, 0:3] + shift_ref[:, 0:3] * inv
    xn_ref[...] = jnp.concatenate([xs, jnp.zeros((n, 5), _F32)], axis=1)


def _update_head_kernel(h_ref, x8_ref, magg_ref, shift_ref, pos8_ref,
                        wh0a_ref, wh0b_ref, bh0_ref, wh1_ref, bh1_ref,
                        wf_ref, bf_ref,
                        hout_ref, vec_ref, *, n):
    inv = _F32(1.0 / (n - 1))
    h = h_ref[...]
    magg = magg_ref[...] * inv
    t = _silu(jnp.dot(h, wh0a_ref[...], preferred_element_type=_F32)
              + jnp.dot(magg, wh0b_ref[...], preferred_element_type=_F32)
              + bh0_ref[...])
    dh = jnp.dot(t, wh1_ref[...], preferred_element_type=_F32) + bh1_ref[...]
    h2 = h + dh
    xs = x8_ref[:, 0:3] + shift_ref[:, 0:3] * inv
    vec3 = xs - pos8_ref[:, 0:3]
    vec_ref[...] = jnp.concatenate([vec3, jnp.zeros((n, 5), _F32)], axis=1)
    z = h2 - jnp.max(h2, axis=1, keepdims=True)
    ez = jnp.exp(z)
    sm = ez / jnp.sum(ez, axis=1, keepdims=True)
    hout_ref[...] = jnp.dot(sm, wf_ref[...], preferred_element_type=_F32) \
        + bf_ref[...]


def kernel(positions, features, We0, be0, We1, be1, Wx0, bx0, Wx1, bx1,
           Wh0, bh0, Wh1, bh1, Wf, bf):
    n = positions.shape[0]
    hdim = features.shape[-1]
    nb = We0.shape[0]
    tr = _TR
    nt = n // tr
    n2 = n // 2
    e2 = tr * n2

    h = features[:, 0, :].astype(_F32)
    x8 = jnp.pad(positions[:, 0, :].astype(_F32), ((0, 0), (0, 5)))
    pos8 = x8

    recv_of_row = jnp.arange(e2, dtype=jnp.int32) // n2
    gt_const = (recv_of_row[:, None]
                == jnp.arange(tr, dtype=jnp.int32)[None, :]).astype(_F32)
    g_const = gt_const.T
    sel16 = ((jnp.arange(16)[:, None] // 8)
             == (jnp.arange(16)[None, :] // 8)).astype(_F32)

    prep_call = pl.pallas_call(
        _prep_kernel,
        out_shape=[
            jax.ShapeDtypeStruct((n, hdim), _F32),
            jax.ShapeDtypeStruct((n, hdim), _F32),
            jax.ShapeDtypeStruct((n, 8), _F32),
            jax.ShapeDtypeStruct((n, 8), _F32),
            jax.ShapeDtypeStruct((n, 8), _F32),
        ],
    )

    full = lambda t: (0, 0)
    tile = lambda t: (t, 0)
    edge_call = pl.pallas_call(
        functools.partial(_edge_kernel, n=n, tr=tr, hdim=hdim),
        grid=(nt,),
        in_specs=[
            pl.BlockSpec((1, n2, 2 * hdim), lambda t: (0, 0, 0)),
            pl.BlockSpec((tr, hdim), tile),
            pl.BlockSpec((tr, hdim), tile),
            pl.BlockSpec((tr, 8), tile),
            pl.BlockSpec((1, n2, 16), lambda t: (0, 0, 0)),
            pl.BlockSpec((1, n2, 16), lambda t: (0, 0, 0)),
            pl.BlockSpec((e2, tr), full),
            pl.BlockSpec((tr, e2), full),
            pl.BlockSpec((2 * hdim, 2 * hdim), full),
            pl.BlockSpec((1, 2 * hdim), full),
            pl.BlockSpec((2 * hdim, 2 * hdim), full),
            pl.BlockSpec((1, 2 * hdim), full),
            pl.BlockSpec((2 * hdim, 16), full),
            pl.BlockSpec((1, 1), full),
            pl.BlockSpec((16, 2 * hdim), full),
            pl.BlockSpec((16, 16), full),
            pl.BlockSpec((hdim, hdim), full),
            pl.BlockSpec((1, hdim), full),
        ],
        out_specs=[
            pl.BlockSpec((tr, hdim), tile),
            pl.BlockSpec((tr, 8), tile),
        ],
        out_shape=[
            jax.ShapeDtypeStruct((n, hdim), _F32),
            jax.ShapeDtypeStruct((n, 8), _F32),
        ],
    )

    update_call = pl.pallas_call(
        functools.partial(_update_kernel, n=n),
        out_shape=[
            jax.ShapeDtypeStruct((n, hdim), _F32),
            jax.ShapeDtypeStruct((n, 8), _F32),
        ],
    )

    head_call = pl.pallas_call(
        functools.partial(_update_head_kernel, n=n),
        out_shape=[
            jax.ShapeDtypeStruct((n, hdim), _F32),
            jax.ShapeDtypeStruct((n, 8), _F32),
        ],
    )

    h_out = None
    vec8 = None
    for b in range(nb):
        wea = We0[b, :hdim]
        web = We0[b, hdim:2 * hdim]
        wsq = We0[b, 2 * hdim]
        zz = jnp.zeros((hdim, hdim), _F32)
        we1bd = jnp.block([[We1[b], zz], [zz, We1[b]]])
        wx0bd = jnp.block([[Wx0[b], zz], [zz, Wx0[b]]])
        wx1sel = jnp.zeros((2 * hdim, 16), _F32)
        wx1sel = wx1sel.at[:hdim, 0:8].set(jnp.broadcast_to(Wx1[b], (hdim, 8)))
        wx1sel = wx1sel.at[hdim:, 8:16].set(jnp.broadcast_to(Wx1[b], (hdim, 8)))
        selw = jnp.zeros((16, 2 * hdim), _F32)
        selw = selw.at[0:8, :hdim].set(jnp.broadcast_to(wsq[None], (8, hdim)))
        selw = selw.at[8:16, hdim:].set(jnp.broadcast_to(wsq[None], (8, hdim)))
        be1d = jnp.concatenate([be1[b], be1[b]])[None]
        bx0d = jnp.concatenate([bx0[b], bx0[b]])[None]

        amat, bmat, pmat, qmat, xmat = prep_call(x8, h, wea, web, be0[b][None])
        a2 = amat.reshape(n2, 2 * hdim)[None]
        q2 = qmat.reshape(n2, 16)[None]
        x2 = xmat.reshape(n2, 16)[None]
        magg, shift = edge_call(a2, amat, bmat, pmat, q2, x2,
                                gt_const, g_const,
                                we1bd, be1d, wx0bd, bx0d,
                                wx1sel, bx1[b][None], selw, sel16,
                                We1[b], be1[b][None])
        if b < nb - 1:
            h, x8 = update_call(h, x8, magg, shift,
                                Wh0[b, :hdim], Wh0[b, hdim:], bh0[b][None],
                                Wh1[b], bh1[b][None])
        else:
            h_out, vec8 = head_call(h, x8, magg, shift, pos8,
                                    Wh0[b, :hdim], Wh0[b, hdim:],
                                    bh0[b][None], Wh1[b], bh1[b][None],
                                    Wf, bf[None])

    vectors = vec8[:, 0:3][:, None, :]
    return vectors, h_out


# speedup vs baseline: 40.2544x; 1.1861x over previous
import functools

import jax
import jax.numpy as jnp
from jax.experimental import pallas as pl

_F32 = jnp.float32
_TR = 8


def _silu(z):
    return z * jax.nn.sigmoid(z)


def _prep_kernel(x8_ref, h_ref, wea_ref, web_ref, be0_ref,
                 a_ref, b_ref, p_ref, q_ref, x_ref):
    x = x8_ref[...]
    h = h_ref[...]
    n = x.shape[0]
    a_ref[...] = jnp.dot(h, wea_ref[...], preferred_element_type=_F32)
    b_ref[...] = jnp.dot(h, web_ref[...], preferred_element_type=_F32) \
        + be0_ref[...]
    x3 = x[:, 0:3]
    nx = jnp.sum(x3 * x3, axis=1, keepdims=True)
    ones = jnp.ones((n, 1), _F32)
    zer3 = jnp.zeros((n, 3), _F32)
    p_ref[...] = jnp.concatenate([x3, nx, ones, zer3], axis=1)
    q_ref[...] = jnp.concatenate([-2.0 * x3, ones, nx, zer3], axis=1)
    x_ref[...] = x


def _edge_kernel(a2_ref, a_ref, b_ref, p_ref, q2_ref, x2_ref,
                 gt_ref, g_ref,
                 we1bd_ref, be1d_ref, wx0bd_ref, bx0d_ref,
                 wx1sel_ref, bx1_ref, selw_ref, sel16_ref,
                 we1_ref, be1_ref,
                 magg_ref, shift_ref, *, n, tr, hdim):
    n2 = n // 2
    e2 = tr * n2

    a_part = jnp.broadcast_to(a2_ref[...], (tr, n2, 2 * hdim)).reshape(
        e2, 2 * hdim)
    qt = jnp.broadcast_to(q2_ref[...], (tr, n2, 16)).reshape(e2, 16)

    gt = gt_ref[...]
    g = g_ref[...]

    bt = b_ref[...]
    pt = p_ref[...]
    b2t = jnp.concatenate([bt, bt], axis=1)
    p2t = jnp.concatenate([pt, pt], axis=1)

    p_part = jnp.dot(gt, p2t, preferred_element_type=_F32)
    pq = qt * p_part

    sq16 = jnp.maximum(
        jnp.dot(pq, sel16_ref[...], preferred_element_type=_F32), 0.0)
    gtpq = jnp.concatenate([gt, pq], axis=1)
    bsel = jnp.concatenate([b2t, selw_ref[...]], axis=0)
    bsq_part = jnp.dot(gtpq, bsel, preferred_element_type=_F32)

    bf16 = jnp.bfloat16
    m0 = _silu(a_part + bsq_part).astype(bf16)
    m1 = _silu(jnp.dot(m0, we1bd_ref[...], preferred_element_type=_F32)
               + be1d_ref[...]).astype(bf16)
    av = _silu(jnp.dot(m1, wx0bd_ref[...], preferred_element_type=_F32)
               + bx0d_ref[...]).astype(bf16)
    coef16 = jnp.dot(av, wx1sel_ref[...], preferred_element_type=_F32) \
        + bx1_ref[...]
    scale16 = coef16 / (jnp.sqrt(sq16) + 1.0)

    maggp = jnp.dot(g, m1, preferred_element_type=_F32)
    m0d = _silu(a_ref[...] + bt)
    m1d = _silu(jnp.dot(m0d, we1_ref[...], preferred_element_type=_F32)
                + be1_ref[...])
    magg_ref[...] = maggp[:, :hdim] + maggp[:, hdim:] - m1d

    xt = jnp.broadcast_to(x2_ref[...], (tr, n2, 16)).reshape(e2, 16)
    sxs = jnp.concatenate([scale16 * xt, scale16], axis=1).astype(bf16)
    sump = jnp.dot(g, sxs, preferred_element_type=_F32)
    part1 = sump[:, 0:8] + sump[:, 8:16]
    ssum = sump[:, 16:17] + sump[:, 24:25]
    shift_ref[...] = part1 - pt * ssum


def _update_kernel(h_ref, x8_ref, magg_ref, shift_ref,
                   wh0a_ref, wh0b_ref, bh0_ref, wh1_ref, bh1_ref,
                   hn_ref, xn_ref, *, n):
    inv = _F32(1.0 / (n - 1))
    h = h_ref[...]
    magg = magg_ref[...] * inv
    t = _silu(jnp.dot(h, wh0a_ref[...], preferred_element_type=_F32)
              + jnp.dot(magg, wh0b_ref[...], preferred_element_type=_F32)
              + bh0_ref[...])
    dh = jnp.dot(t, wh1_ref[...], preferred_element_type=_F32) + bh1_ref[...]
    hn_ref[...] = h + dh
    xs = x8_ref[:, 0:3] + shift_ref[:, 0:3] * inv
    xn_ref[...] = jnp.concatenate([xs, jnp.zeros((n, 5), _F32)], axis=1)


def _update_head_kernel(h_ref, x8_ref, magg_ref, shift_ref, pos8_ref,
                        wh0a_ref, wh0b_ref, bh0_ref, wh1_ref, bh1_ref,
                        wf_ref, bf_ref,
                        hout_ref, vec_ref, *, n):
    inv = _F32(1.0 / (n - 1))
    h = h_ref[...]
    magg = magg_ref[...] * inv
    t = _silu(jnp.dot(h, wh0a_ref[...], preferred_element_type=_F32)
              + jnp.dot(magg, wh0b_ref[...], preferred_element_type=_F32)
              + bh0_ref[...])
    dh = jnp.dot(t, wh1_ref[...], preferred_element_type=_F32) + bh1_ref[...]
    h2 = h + dh
    xs = x8_ref[:, 0:3] + shift_ref[:, 0:3] * inv
    vec3 = xs - pos8_ref[:, 0:3]
    vec_ref[...] = jnp.concatenate([vec3, jnp.zeros((n, 5), _F32)], axis=1)
    z = h2 - jnp.max(h2, axis=1, keepdims=True)
    ez = jnp.exp(z)
    sm = ez / jnp.sum(ez, axis=1, keepdims=True)
    hout_ref[...] = jnp.dot(sm, wf_ref[...], preferred_element_type=_F32) \
        + bf_ref[...]


def kernel(positions, features, We0, be0, We1, be1, Wx0, bx0, Wx1, bx1,
           Wh0, bh0, Wh1, bh1, Wf, bf):
    n = positions.shape[0]
    hdim = features.shape[-1]
    nb = We0.shape[0]
    tr = _TR
    nt = n // tr
    n2 = n // 2
    e2 = tr * n2

    h = features[:, 0, :].astype(_F32)
    x8 = jnp.pad(positions[:, 0, :].astype(_F32), ((0, 0), (0, 5)))
    pos8 = x8

    recv_of_row = jnp.arange(e2, dtype=jnp.int32) // n2
    gt_const = (recv_of_row[:, None]
                == jnp.arange(tr, dtype=jnp.int32)[None, :]).astype(_F32)
    g_const = gt_const.T.astype(jnp.bfloat16)
    sel16 = ((jnp.arange(16)[:, None] // 8)
             == (jnp.arange(16)[None, :] // 8)).astype(_F32)

    prep_call = pl.pallas_call(
        _prep_kernel,
        out_shape=[
            jax.ShapeDtypeStruct((n, hdim), _F32),
            jax.ShapeDtypeStruct((n, hdim), _F32),
            jax.ShapeDtypeStruct((n, 8), _F32),
            jax.ShapeDtypeStruct((n, 8), _F32),
            jax.ShapeDtypeStruct((n, 8), _F32),
        ],
    )

    full = lambda t: (0, 0)
    tile = lambda t: (t, 0)
    edge_call = pl.pallas_call(
        functools.partial(_edge_kernel, n=n, tr=tr, hdim=hdim),
        grid=(nt,),
        in_specs=[
            pl.BlockSpec((1, n2, 2 * hdim), lambda t: (0, 0, 0)),
            pl.BlockSpec((tr, hdim), tile),
            pl.BlockSpec((tr, hdim), tile),
            pl.BlockSpec((tr, 8), tile),
            pl.BlockSpec((1, n2, 16), lambda t: (0, 0, 0)),
            pl.BlockSpec((1, n2, 16), lambda t: (0, 0, 0)),
            pl.BlockSpec((e2, tr), full),
            pl.BlockSpec((tr, e2), full),
            pl.BlockSpec((2 * hdim, 2 * hdim), full),
            pl.BlockSpec((1, 2 * hdim), full),
            pl.BlockSpec((2 * hdim, 2 * hdim), full),
            pl.BlockSpec((1, 2 * hdim), full),
            pl.BlockSpec((2 * hdim, 16), full),
            pl.BlockSpec((1, 1), full),
            pl.BlockSpec((16, 2 * hdim), full),
            pl.BlockSpec((16, 16), full),
            pl.BlockSpec((hdim, hdim), full),
            pl.BlockSpec((1, hdim), full),
        ],
        out_specs=[
            pl.BlockSpec((tr, hdim), tile),
            pl.BlockSpec((tr, 8), tile),
        ],
        out_shape=[
            jax.ShapeDtypeStruct((n, hdim), _F32),
            jax.ShapeDtypeStruct((n, 8), _F32),
        ],
    )

    update_call = pl.pallas_call(
        functools.partial(_update_kernel, n=n),
        out_shape=[
            jax.ShapeDtypeStruct((n, hdim), _F32),
            jax.ShapeDtypeStruct((n, 8), _F32),
        ],
    )

    head_call = pl.pallas_call(
        functools.partial(_update_head_kernel, n=n),
        out_shape=[
            jax.ShapeDtypeStruct((n, hdim), _F32),
            jax.ShapeDtypeStruct((n, 8), _F32),
        ],
    )

    h_out = None
    vec8 = None
    for b in range(nb):
        wea = We0[b, :hdim]
        web = We0[b, hdim:2 * hdim]
        wsq = We0[b, 2 * hdim]
        zz = jnp.zeros((hdim, hdim), _F32)
        we1bd = jnp.block([[We1[b], zz], [zz, We1[b]]]).astype(jnp.bfloat16)
        wx0bd = jnp.block([[Wx0[b], zz], [zz, Wx0[b]]]).astype(jnp.bfloat16)
        wx1sel = jnp.zeros((2 * hdim, 16), _F32)
        wx1sel = wx1sel.at[:hdim, 0:8].set(jnp.broadcast_to(Wx1[b], (hdim, 8)))
        wx1sel = wx1sel.at[hdim:, 8:16].set(jnp.broadcast_to(Wx1[b], (hdim, 8)))
        wx1sel = wx1sel.astype(jnp.bfloat16)
        selw = jnp.zeros((16, 2 * hdim), _F32)
        selw = selw.at[0:8, :hdim].set(jnp.broadcast_to(wsq[None], (8, hdim)))
        selw = selw.at[8:16, hdim:].set(jnp.broadcast_to(wsq[None], (8, hdim)))
        be1d = jnp.concatenate([be1[b], be1[b]])[None]
        bx0d = jnp.concatenate([bx0[b], bx0[b]])[None]

        amat, bmat, pmat, qmat, xmat = prep_call(x8, h, wea, web, be0[b][None])
        a2 = amat.reshape(n2, 2 * hdim)[None]
        q2 = qmat.reshape(n2, 16)[None]
        x2 = xmat.reshape(n2, 16)[None]
        magg, shift = edge_call(a2, amat, bmat, pmat, q2, x2,
                                gt_const, g_const,
                                we1bd, be1d, wx0bd, bx0d,
                                wx1sel, bx1[b][None], selw, sel16,
                                We1[b], be1[b][None])
        if b < nb - 1:
            h, x8 = update_call(h, x8, magg, shift,
                                Wh0[b, :hdim], Wh0[b, hdim:], bh0[b][None],
                                Wh1[b], bh1[b][None])
        else:
            h_out, vec8 = head_call(h, x8, magg, shift, pos8,
                                    Wh0[b, :hdim], Wh0[b, hdim:],
                                    bh0[b][None], Wh1[b], bh1[b][None],
                                    Wf, bf[None])

    vectors = vec8[:, 0:3][:, None, :]
    return vectors, h_out


# speedup vs baseline: 44.0498x; 1.0943x over previous
import functools

import jax
import jax.numpy as jnp
from jax.experimental import pallas as pl

_F32 = jnp.float32
_TR = 8


def _silu(z):
    t = 0.5 * z
    return t * jnp.tanh(t) + t


def _prep_kernel(x8_ref, h_ref, wea_ref, web_ref, be0_ref,
                 a_ref, b_ref, p_ref, q_ref, x_ref):
    x = x8_ref[...]
    h = h_ref[...]
    n = x.shape[0]
    a_ref[...] = jnp.dot(h, wea_ref[...], preferred_element_type=_F32)
    b_ref[...] = jnp.dot(h, web_ref[...], preferred_element_type=_F32) \
        + be0_ref[...]
    x3 = x[:, 0:3]
    nx = jnp.sum(x3 * x3, axis=1, keepdims=True)
    ones = jnp.ones((n, 1), _F32)
    zer3 = jnp.zeros((n, 3), _F32)
    p_ref[...] = jnp.concatenate([x3, nx, ones, zer3], axis=1)
    q_ref[...] = jnp.concatenate([-2.0 * x3, ones, nx, zer3], axis=1)
    x_ref[...] = x


def _edge_kernel(a2_ref, a_ref, b_ref, p_ref, q2_ref, x2_ref,
                 gt_ref, g_ref,
                 we1bd_ref, be1d_ref, wx0bd_ref, bx0d_ref,
                 wx1sel_ref, bx1_ref, selw_ref, sel16_ref,
                 we1_ref, be1_ref,
                 magg_ref, shift_ref, *, n, tr, hdim):
    n2 = n // 2
    e2 = tr * n2

    a_part = jnp.broadcast_to(a2_ref[...], (tr, n2, 2 * hdim)).reshape(
        e2, 2 * hdim)
    qt = jnp.broadcast_to(q2_ref[...], (tr, n2, 16)).reshape(e2, 16)

    gt = gt_ref[...]
    g = g_ref[...]

    bt = b_ref[...]
    pt = p_ref[...]
    b2t = jnp.concatenate([bt, bt], axis=1)
    p2t = jnp.concatenate([pt, pt], axis=1)

    p_part = jnp.dot(gt, p2t, preferred_element_type=_F32)
    pq = qt * p_part

    sq16 = jnp.maximum(
        jnp.dot(pq, sel16_ref[...], preferred_element_type=_F32), 0.0)
    gtpq = jnp.concatenate([gt, pq], axis=1)
    bsel = jnp.concatenate([b2t, selw_ref[...]], axis=0)
    bsq_part = jnp.dot(gtpq, bsel, preferred_element_type=_F32)

    bf16 = jnp.bfloat16
    m0 = _silu(a_part + bsq_part).astype(bf16)
    m1 = _silu(jnp.dot(m0, we1bd_ref[...], preferred_element_type=_F32)
               + be1d_ref[...]).astype(bf16)
    av = _silu(jnp.dot(m1, wx0bd_ref[...], preferred_element_type=_F32)
               + bx0d_ref[...]).astype(bf16)
    coef16 = jnp.dot(av, wx1sel_ref[...], preferred_element_type=_F32) \
        + bx1_ref[...]
    scale16 = coef16 / (jnp.sqrt(sq16) + 1.0)

    maggp = jnp.dot(g, m1, preferred_element_type=_F32)
    m0d = _silu(a_ref[...] + bt)
    m1d = _silu(jnp.dot(m0d, we1_ref[...], preferred_element_type=_F32)
                + be1_ref[...])
    magg_ref[...] = maggp[:, :hdim] + maggp[:, hdim:] - m1d

    xt = jnp.broadcast_to(x2_ref[...], (tr, n2, 16)).reshape(e2, 16)
    sxs = jnp.concatenate([scale16 * xt, scale16], axis=1).astype(bf16)
    sump = jnp.dot(g, sxs, preferred_element_type=_F32)
    part1 = sump[:, 0:8] + sump[:, 8:16]
    ssum = sump[:, 16:17] + sump[:, 24:25]
    shift_ref[...] = part1 - pt * ssum


def _update_kernel(h_ref, x8_ref, magg_ref, shift_ref,
                   wh0a_ref, wh0b_ref, bh0_ref, wh1_ref, bh1_ref,
                   hn_ref, xn_ref, *, n):
    inv = _F32(1.0 / (n - 1))
    h = h_ref[...]
    magg = magg_ref[...] * inv
    t = _silu(jnp.dot(h, wh0a_ref[...], preferred_element_type=_F32)
              + jnp.dot(magg, wh0b_ref[...], preferred_element_type=_F32)
              + bh0_ref[...])
    dh = jnp.dot(t, wh1_ref[...], preferred_element_type=_F32) + bh1_ref[...]
    hn_ref[...] = h + dh
    xs = x8_ref[:, 0:3] + shift_ref[:, 0:3] * inv
    xn_ref[...] = jnp.concatenate([xs, jnp.zeros((n, 5), _F32)], axis=1)


def _update_head_kernel(h_ref, x8_ref, magg_ref, shift_ref, pos8_ref,
                        wh0a_ref, wh0b_ref, bh0_ref, wh1_ref, bh1_ref,
                        wf_ref, bf_ref,
                        hout_ref, vec_ref, *, n):
    inv = _F32(1.0 / (n - 1))
    h = h_ref[...]
    magg = magg_ref[...] * inv
    t = _silu(jnp.dot(h, wh0a_ref[...], preferred_element_type=_F32)
              + jnp.dot(magg, wh0b_ref[...], preferred_element_type=_F32)
              + bh0_ref[...])
    dh = jnp.dot(t, wh1_ref[...], preferred_element_type=_F32) + bh1_ref[...]
    h2 = h + dh
    xs = x8_ref[:, 0:3] + shift_ref[:, 0:3] * inv
    vec3 = xs - pos8_ref[:, 0:3]
    vec_ref[...] = jnp.concatenate([vec3, jnp.zeros((n, 5), _F32)], axis=1)
    z = h2 - jnp.max(h2, axis=1, keepdims=True)
    ez = jnp.exp(z)
    sm = ez / jnp.sum(ez, axis=1, keepdims=True)
    hout_ref[...] = jnp.dot(sm, wf_ref[...], preferred_element_type=_F32) \
        + bf_ref[...]


def kernel(positions, features, We0, be0, We1, be1, Wx0, bx0, Wx1, bx1,
           Wh0, bh0, Wh1, bh1, Wf, bf):
    n = positions.shape[0]
    hdim = features.shape[-1]
    nb = We0.shape[0]
    tr = _TR
    nt = n // tr
    n2 = n // 2
    e2 = tr * n2

    h = features[:, 0, :].astype(_F32)
    x8 = jnp.pad(positions[:, 0, :].astype(_F32), ((0, 0), (0, 5)))
    pos8 = x8

    recv_of_row = jnp.arange(e2, dtype=jnp.int32) // n2
    gt_const = (recv_of_row[:, None]
                == jnp.arange(tr, dtype=jnp.int32)[None, :]).astype(_F32)
    g_const = gt_const.T.astype(jnp.bfloat16)
    sel16 = ((jnp.arange(16)[:, None] // 8)
             == (jnp.arange(16)[None, :] // 8)).astype(_F32)

    prep_call = pl.pallas_call(
        _prep_kernel,
        out_shape=[
            jax.ShapeDtypeStruct((n, hdim), _F32),
            jax.ShapeDtypeStruct((n, hdim), _F32),
            jax.ShapeDtypeStruct((n, 8), _F32),
            jax.ShapeDtypeStruct((n, 8), _F32),
            jax.ShapeDtypeStruct((n, 8), _F32),
        ],
    )

    full = lambda t: (0, 0)
    tile = lambda t: (t, 0)
    edge_call = pl.pallas_call(
        functools.partial(_edge_kernel, n=n, tr=tr, hdim=hdim),
        grid=(nt,),
        in_specs=[
            pl.BlockSpec((1, n2, 2 * hdim), lambda t: (0, 0, 0)),
            pl.BlockSpec((tr, hdim), tile),
            pl.BlockSpec((tr, hdim), tile),
            pl.BlockSpec((tr, 8), tile),
            pl.BlockSpec((1, n2, 16), lambda t: (0, 0, 0)),
            pl.BlockSpec((1, n2, 16), lambda t: (0, 0, 0)),
            pl.BlockSpec((e2, tr), full),
            pl.BlockSpec((tr, e2), full),
            pl.BlockSpec((2 * hdim, 2 * hdim), full),
            pl.BlockSpec((1, 2 * hdim), full),
            pl.BlockSpec((2 * hdim, 2 * hdim), full),
            pl.BlockSpec((1, 2 * hdim), full),
            pl.BlockSpec((2 * hdim, 16), full),
            pl.BlockSpec((1, 1), full),
            pl.BlockSpec((16, 2 * hdim), full),
            pl.BlockSpec((16, 16), full),
            pl.BlockSpec((hdim, hdim), full),
            pl.BlockSpec((1, hdim), full),
        ],
        out_specs=[
            pl.BlockSpec((tr, hdim), tile),
            pl.BlockSpec((tr, 8), tile),
        ],
        out_shape=[
            jax.ShapeDtypeStruct((n, hdim), _F32),
            jax.ShapeDtypeStruct((n, 8), _F32),
        ],
    )

    update_call = pl.pallas_call(
        functools.partial(_update_kernel, n=n),
        out_shape=[
            jax.ShapeDtypeStruct((n, hdim), _F32),
            jax.ShapeDtypeStruct((n, 8), _F32),
        ],
    )

    head_call = pl.pallas_call(
        functools.partial(_update_head_kernel, n=n),
        out_shape=[
            jax.ShapeDtypeStruct((n, hdim), _F32),
            jax.ShapeDtypeStruct((n, 8), _F32),
        ],
    )

    h_out = None
    vec8 = None
    for b in range(nb):
        wea = We0[b, :hdim]
        web = We0[b, hdim:2 * hdim]
        wsq = We0[b, 2 * hdim]
        zz = jnp.zeros((hdim, hdim), _F32)
        we1bd = jnp.block([[We1[b], zz], [zz, We1[b]]]).astype(jnp.bfloat16)
        wx0bd = jnp.block([[Wx0[b], zz], [zz, Wx0[b]]]).astype(jnp.bfloat16)
        wx1sel = jnp.zeros((2 * hdim, 16), _F32)
        wx1sel = wx1sel.at[:hdim, 0:8].set(jnp.broadcast_to(Wx1[b], (hdim, 8)))
        wx1sel = wx1sel.at[hdim:, 8:16].set(jnp.broadcast_to(Wx1[b], (hdim, 8)))
        wx1sel = wx1sel.astype(jnp.bfloat16)
        selw = jnp.zeros((16, 2 * hdim), _F32)
        selw = selw.at[0:8, :hdim].set(jnp.broadcast_to(wsq[None], (8, hdim)))
        selw = selw.at[8:16, hdim:].set(jnp.broadcast_to(wsq[None], (8, hdim)))
        be1d = jnp.concatenate([be1[b], be1[b]])[None]
        bx0d = jnp.concatenate([bx0[b], bx0[b]])[None]

        amat, bmat, pmat, qmat, xmat = prep_call(x8, h, wea, web, be0[b][None])
        a2 = amat.reshape(n2, 2 * hdim)[None]
        q2 = qmat.reshape(n2, 16)[None]
        x2 = xmat.reshape(n2, 16)[None]
        magg, shift = edge_call(a2, amat, bmat, pmat, q2, x2,
                                gt_const, g_const,
                                we1bd, be1d, wx0bd, bx0d,
                                wx1sel, bx1[b][None], selw, sel16,
                                We1[b], be1[b][None])
        if b < nb - 1:
            h, x8 = update_call(h, x8, magg, shift,
                                Wh0[b, :hdim], Wh0[b, hdim:], bh0[b][None],
                                Wh1[b], bh1[b][None])
        else:
            h_out, vec8 = head_call(h, x8, magg, shift, pos8,
                                    Wh0[b, :hdim], Wh0[b, hdim:],
                                    bh0[b][None], Wh1[b], bh1[b][None],
                                    Wf, bf[None])

    vectors = vec8[:, 0:3][:, None, :]
    return vectors, h_out


# speedup vs baseline: 45.6629x; 1.0366x over previous
import functools

import jax
import jax.numpy as jnp
from jax.experimental import pallas as pl

_F32 = jnp.float32
_TR = 16


def _silu(z):
    t = 0.5 * z
    return t * jnp.tanh(t) + t


def _prep_kernel(x8_ref, h_ref, wea_ref, web_ref, be0_ref,
                 a_ref, b_ref, p_ref, q_ref, x_ref):
    x = x8_ref[...]
    h = h_ref[...]
    n = x.shape[0]
    a_ref[...] = jnp.dot(h, wea_ref[...], preferred_element_type=_F32)
    b_ref[...] = jnp.dot(h, web_ref[...], preferred_element_type=_F32) \
        + be0_ref[...]
    x3 = x[:, 0:3]
    nx = jnp.sum(x3 * x3, axis=1, keepdims=True)
    ones = jnp.ones((n, 1), _F32)
    zer3 = jnp.zeros((n, 3), _F32)
    p_ref[...] = jnp.concatenate([x3, nx, ones, zer3], axis=1)
    q_ref[...] = jnp.concatenate([-2.0 * x3, ones, nx, zer3], axis=1)
    x_ref[...] = x


def _edge_kernel(a2_ref, a_ref, b_ref, p_ref, q2_ref, x2_ref,
                 gt_ref, g_ref,
                 we1bd_ref, be1d_ref, wx0bd_ref, bx0d_ref,
                 wx1sel_ref, bx1_ref, selw_ref, sel16_ref,
                 we1_ref, be1_ref,
                 magg_ref, shift_ref, *, n, tr, hdim):
    n2 = n // 2
    e2 = tr * n2

    a_part = jnp.broadcast_to(a2_ref[...], (tr, n2, 2 * hdim)).reshape(
        e2, 2 * hdim)
    qt = jnp.broadcast_to(q2_ref[...], (tr, n2, 16)).reshape(e2, 16)

    gt = gt_ref[...]
    g = g_ref[...]

    bt = b_ref[...]
    pt = p_ref[...]
    b2t = jnp.concatenate([bt, bt], axis=1)
    p2t = jnp.concatenate([pt, pt], axis=1)

    p_part = jnp.dot(gt, p2t, preferred_element_type=_F32)
    pq = qt * p_part

    sq16 = jnp.maximum(
        jnp.dot(pq, sel16_ref[...], preferred_element_type=_F32), 0.0)
    gtpq = jnp.concatenate([gt, pq], axis=1)
    bsel = jnp.concatenate([b2t, selw_ref[...]], axis=0)
    bsq_part = jnp.dot(gtpq, bsel, preferred_element_type=_F32)

    bf16 = jnp.bfloat16
    m0 = _silu(a_part + bsq_part).astype(bf16)
    m1 = _silu(jnp.dot(m0, we1bd_ref[...], preferred_element_type=_F32)
               + be1d_ref[...]).astype(bf16)
    av = _silu(jnp.dot(m1, wx0bd_ref[...], preferred_element_type=_F32)
               + bx0d_ref[...]).astype(bf16)
    coef16 = jnp.dot(av, wx1sel_ref[...], preferred_element_type=_F32) \
        + bx1_ref[...]
    scale16 = coef16 / (jnp.sqrt(sq16) + 1.0)

    maggp = jnp.dot(g, m1, preferred_element_type=_F32)
    m0d = _silu(a_ref[...] + bt)
    m1d = _silu(jnp.dot(m0d, we1_ref[...], preferred_element_type=_F32)
                + be1_ref[...])
    magg_ref[...] = maggp[:, :hdim] + maggp[:, hdim:] - m1d

    xt = jnp.broadcast_to(x2_ref[...], (tr, n2, 16)).reshape(e2, 16)
    sxs = jnp.concatenate([scale16 * xt, scale16], axis=1).astype(bf16)
    sump = jnp.dot(g, sxs, preferred_element_type=_F32)
    part1 = sump[:, 0:8] + sump[:, 8:16]
    ssum = sump[:, 16:17] + sump[:, 24:25]
    shift_ref[...] = part1 - pt * ssum


def _update_kernel(h_ref, x8_ref, magg_ref, shift_ref,
                   wh0a_ref, wh0b_ref, bh0_ref, wh1_ref, bh1_ref,
                   hn_ref, xn_ref, *, n):
    inv = _F32(1.0 / (n - 1))
    h = h_ref[...]
    magg = magg_ref[...] * inv
    t = _silu(jnp.dot(h, wh0a_ref[...], preferred_element_type=_F32)
              + jnp.dot(magg, wh0b_ref[...], preferred_element_type=_F32)
              + bh0_ref[...])
    dh = jnp.dot(t, wh1_ref[...], preferred_element_type=_F32) + bh1_ref[...]
    hn_ref[...] = h + dh
    xs = x8_ref[:, 0:3] + shift_ref[:, 0:3] * inv
    xn_ref[...] = jnp.concatenate([xs, jnp.zeros((n, 5), _F32)], axis=1)


def _update_head_kernel(h_ref, x8_ref, magg_ref, shift_ref, pos8_ref,
                        wh0a_ref, wh0b_ref, bh0_ref, wh1_ref, bh1_ref,
                        wf_ref, bf_ref,
                        hout_ref, vec_ref, *, n):
    inv = _F32(1.0 / (n - 1))
    h = h_ref[...]
    magg = magg_ref[...] * inv
    t = _silu(jnp.dot(h, wh0a_ref[...], preferred_element_type=_F32)
              + jnp.dot(magg, wh0b_ref[...], preferred_element_type=_F32)
              + bh0_ref[...])
    dh = jnp.dot(t, wh1_ref[...], preferred_element_type=_F32) + bh1_ref[...]
    h2 = h + dh
    xs = x8_ref[:, 0:3] + shift_ref[:, 0:3] * inv
    vec3 = xs - pos8_ref[:, 0:3]
    vec_ref[...] = jnp.concatenate([vec3, jnp.zeros((n, 5), _F32)], axis=1)
    z = h2 - jnp.max(h2, axis=1, keepdims=True)
    ez = jnp.exp(z)
    sm = ez / jnp.sum(ez, axis=1, keepdims=True)
    hout_ref[...] = jnp.dot(sm, wf_ref[...], preferred_element_type=_F32) \
        + bf_ref[...]


def kernel(positions, features, We0, be0, We1, be1, Wx0, bx0, Wx1, bx1,
           Wh0, bh0, Wh1, bh1, Wf, bf):
    n = positions.shape[0]
    hdim = features.shape[-1]
    nb = We0.shape[0]
    tr = _TR
    nt = n // tr
    n2 = n // 2
    e2 = tr * n2

    h = features[:, 0, :].astype(_F32)
    x8 = jnp.pad(positions[:, 0, :].astype(_F32), ((0, 0), (0, 5)))
    pos8 = x8

    recv_of_row = jnp.arange(e2, dtype=jnp.int32) // n2
    gt_const = (recv_of_row[:, None]
                == jnp.arange(tr, dtype=jnp.int32)[None, :]).astype(_F32)
    g_const = gt_const.T.astype(jnp.bfloat16)
    sel16 = ((jnp.arange(16)[:, None] // 8)
             == (jnp.arange(16)[None, :] // 8)).astype(_F32)

    prep_call = pl.pallas_call(
        _prep_kernel,
        out_shape=[
            jax.ShapeDtypeStruct((n, hdim), _F32),
            jax.ShapeDtypeStruct((n, hdim), _F32),
            jax.ShapeDtypeStruct((n, 8), _F32),
            jax.ShapeDtypeStruct((n, 8), _F32),
            jax.ShapeDtypeStruct((n, 8), _F32),
        ],
    )

    full = lambda t: (0, 0)
    tile = lambda t: (t, 0)
    edge_call = pl.pallas_call(
        functools.partial(_edge_kernel, n=n, tr=tr, hdim=hdim),
        grid=(nt,),
        in_specs=[
            pl.BlockSpec((1, n2, 2 * hdim), lambda t: (0, 0, 0)),
            pl.BlockSpec((tr, hdim), tile),
            pl.BlockSpec((tr, hdim), tile),
            pl.BlockSpec((tr, 8), tile),
            pl.BlockSpec((1, n2, 16), lambda t: (0, 0, 0)),
            pl.BlockSpec((1, n2, 16), lambda t: (0, 0, 0)),
            pl.BlockSpec((e2, tr), full),
            pl.BlockSpec((tr, e2), full),
            pl.BlockSpec((2 * hdim, 2 * hdim), full),
            pl.BlockSpec((1, 2 * hdim), full),
            pl.BlockSpec((2 * hdim, 2 * hdim), full),
            pl.BlockSpec((1, 2 * hdim), full),
            pl.BlockSpec((2 * hdim, 16), full),
            pl.BlockSpec((1, 1), full),
            pl.BlockSpec((16, 2 * hdim), full),
            pl.BlockSpec((16, 16), full),
            pl.BlockSpec((hdim, hdim), full),
            pl.BlockSpec((1, hdim), full),
        ],
        out_specs=[
            pl.BlockSpec((tr, hdim), tile),
            pl.BlockSpec((tr, 8), tile),
        ],
        out_shape=[
            jax.ShapeDtypeStruct((n, hdim), _F32),
            jax.ShapeDtypeStruct((n, 8), _F32),
        ],
    )

    update_call = pl.pallas_call(
        functools.partial(_update_kernel, n=n),
        out_shape=[
            jax.ShapeDtypeStruct((n, hdim), _F32),
            jax.ShapeDtypeStruct((n, 8), _F32),
        ],
    )

    head_call = pl.pallas_call(
        functools.partial(_update_head_kernel, n=n),
        out_shape=[
            jax.ShapeDtypeStruct((n, hdim), _F32),
            jax.ShapeDtypeStruct((n, 8), _F32),
        ],
    )

    h_out = None
    vec8 = None
    for b in range(nb):
        wea = We0[b, :hdim]
        web = We0[b, hdim:2 * hdim]
        wsq = We0[b, 2 * hdim]
        zz = jnp.zeros((hdim, hdim), _F32)
        we1bd = jnp.block([[We1[b], zz], [zz, We1[b]]]).astype(jnp.bfloat16)
        wx0bd = jnp.block([[Wx0[b], zz], [zz, Wx0[b]]]).astype(jnp.bfloat16)
        wx1sel = jnp.zeros((2 * hdim, 16), _F32)
        wx1sel = wx1sel.at[:hdim, 0:8].set(jnp.broadcast_to(Wx1[b], (hdim, 8)))
        wx1sel = wx1sel.at[hdim:, 8:16].set(jnp.broadcast_to(Wx1[b], (hdim, 8)))
        wx1sel = wx1sel.astype(jnp.bfloat16)
        selw = jnp.zeros((16, 2 * hdim), _F32)
        selw = selw.at[0:8, :hdim].set(jnp.broadcast_to(wsq[None], (8, hdim)))
        selw = selw.at[8:16, hdim:].set(jnp.broadcast_to(wsq[None], (8, hdim)))
        be1d = jnp.concatenate([be1[b], be1[b]])[None]
        bx0d = jnp.concatenate([bx0[b], bx0[b]])[None]

        amat, bmat, pmat, qmat, xmat = prep_call(x8, h, wea, web, be0[b][None])
        a2 = amat.reshape(n2, 2 * hdim)[None]
        q2 = qmat.reshape(n2, 16)[None]
        x2 = xmat.reshape(n2, 16)[None]
        magg, shift = edge_call(a2, amat, bmat, pmat, q2, x2,
                                gt_const, g_const,
                                we1bd, be1d, wx0bd, bx0d,
                                wx1sel, bx1[b][None], selw, sel16,
                                We1[b], be1[b][None])
        if b < nb - 1:
            h, x8 = update_call(h, x8, magg, shift,
                                Wh0[b, :hdim], Wh0[b, hdim:], bh0[b][None],
                                Wh1[b], bh1[b][None])
        else:
            h_out, vec8 = head_call(h, x8, magg, shift, pos8,
                                    Wh0[b, :hdim], Wh0[b, hdim:],
                                    bh0[b][None], Wh1[b], bh1[b][None],
                                    Wf, bf[None])

    vectors = vec8[:, 0:3][:, None, :]
    return vectors, h_out
